# baseline (speedup 1.0000x reference)
import dataclasses
import functools

import jax
import jax.numpy as jnp
from jax import lax
from jax.experimental import pallas as pl
from jax.experimental.pallas import tpu as pltpu
from jax.experimental.pallas import tpu_sc as plsc

R = 4096
C = 1000
K = 200
L = 16
NW = 32
ROWS_PER_W = R // NW
BLK = 16
NB = ROWS_PER_W // BLK
CHUNK_OFFS = tuple(range(0, K - L + 1, L)) + (K - L,)


def _sc_gather(x, idx):
    mesh = plsc.VectorSubcoreMesh(core_axis_name="c", subcore_axis_name="s")
    cp = pltpu.CompilerParams()
    if "needs_layout_passes" in pltpu.CompilerParams.__dataclass_fields__:
        cp = dataclasses.replace(cp, needs_layout_passes=False)
    cp = dataclasses.replace(cp, use_tc_tiling_on_sc=True)

    @functools.partial(
        pl.kernel,
        out_type=jax.ShapeDtypeStruct((R, K), jnp.float32),
        mesh=mesh,
        compiler_params=cp,
        scratch_types=[
            pltpu.VMEM((BLK, C), jnp.float32),
            pltpu.VMEM((BLK, C), jnp.float32),
            pltpu.VMEM((ROWS_PER_W, K), jnp.int32),
            pltpu.VMEM((ROWS_PER_W, K), jnp.float32),
            pltpu.SemaphoreType.DMA,
            pltpu.SemaphoreType.DMA,
        ],
    )
    def k(x_hbm, i_hbm, o_hbm, xv0, xv1, iv, ov, sx0, sx1):
        wid = lax.axis_index("s") * 2 + lax.axis_index("c")
        row0 = wid * ROWS_PER_W

        def x_copy(b, buf, sem):
            return pltpu.make_async_copy(
                x_hbm.at[pl.ds(row0 + b * BLK, BLK)], buf, sem)

        def gather_block(b, buf):
            @pl.loop(0, BLK)
            def _(r):
                rvec = jnp.full((L,), 0, jnp.int32) + r
                orow = b * BLK + r
                for off in CHUNK_OFFS:
                    cols = iv[orow, pl.ds(off, L)]
                    ov[orow, pl.ds(off, L)] = plsc.load_gather(buf, [rvec, cols])

        x_copy(0, xv0, sx0).start()
        pltpu.sync_copy(i_hbm.at[pl.ds(row0, ROWS_PER_W)], iv)

        @pl.loop(0, NB // 2)
        def _(g):
            b0 = 2 * g
            x_copy(b0 + 1, xv1, sx1).start()
            x_copy(b0, xv0, sx0).wait()
            gather_block(b0, xv0)

            @pl.when(b0 + 2 < NB)
            def _():
                x_copy(b0 + 2, xv0, sx0).start()

            x_copy(b0 + 1, xv1, sx1).wait()
            gather_block(b0 + 1, xv1)

        pltpu.sync_copy(ov, o_hbm.at[pl.ds(row0, ROWS_PER_W)])

    return k(x, idx)


def kernel(x, indices):
    return _sc_gather(x, indices.astype(jnp.int32))


# speedup vs baseline: 1.0355x; 1.0355x over previous
import dataclasses
import functools

import jax
import jax.numpy as jnp
from jax import lax
from jax.experimental import pallas as pl
from jax.experimental.pallas import tpu as pltpu
from jax.experimental.pallas import tpu_sc as plsc

R = 4096
C = 1000
K = 200
L = 16
NW = 32
ROWS_PER_W = R // NW
BLK = 16
NB = ROWS_PER_W // BLK
JU = 8


def _sc_gather_t(x, idxt):
    mesh = plsc.VectorSubcoreMesh(core_axis_name="c", subcore_axis_name="s")
    cp = pltpu.CompilerParams()
    if "needs_layout_passes" in pltpu.CompilerParams.__dataclass_fields__:
        cp = dataclasses.replace(cp, needs_layout_passes=False)

    @functools.partial(
        pl.kernel,
        out_type=jax.ShapeDtypeStruct((K, R), jnp.float32),
        mesh=mesh,
        compiler_params=cp,
        scratch_types=[
            pltpu.VMEM((BLK, C), jnp.float32),
            pltpu.VMEM((BLK, C), jnp.float32),
            pltpu.VMEM((K, ROWS_PER_W), jnp.int32),
            pltpu.VMEM((K, ROWS_PER_W), jnp.float32),
            pltpu.SemaphoreType.DMA,
            pltpu.SemaphoreType.DMA,
        ],
    )
    def k(x_hbm, i_hbm, o_hbm, xv0, xv1, iv, ov, sx0, sx1):
        wid = lax.axis_index("s") * 2 + lax.axis_index("c")
        row0 = wid * ROWS_PER_W
        iota = lax.iota(jnp.int32, L)

        def x_cp(b, buf, sem):
            return pltpu.make_async_copy(
                x_hbm.at[pl.ds(row0 + b * BLK, BLK)], buf, sem)

        def gather_block(b, buf):
            @pl.loop(0, K // JU)
            def _(jb):
                j0 = jb * JU
                for dj in range(JU):
                    cols = iv[j0 + dj, pl.ds(b * BLK, L)]
                    ov[j0 + dj, pl.ds(b * BLK, L)] = plsc.load_gather(
                        buf, [iota, cols])

        x_cp(0, xv0, sx0).start()
        pltpu.sync_copy(i_hbm.at[pl.ds(0, K), pl.ds(row0, ROWS_PER_W)], iv)

        @pl.loop(0, NB // 2)
        def _(g):
            b0 = 2 * g
            x_cp(b0 + 1, xv1, sx1).start()
            x_cp(b0, xv0, sx0).wait()
            gather_block(b0, xv0)

            @pl.when(b0 + 2 < NB)
            def _():
                x_cp(b0 + 2, xv0, sx0).start()

            x_cp(b0 + 1, xv1, sx1).wait()
            gather_block(b0 + 1, xv1)

        pltpu.sync_copy(ov, o_hbm.at[pl.ds(0, K), pl.ds(row0, ROWS_PER_W)])

    return k(x, idxt)


def kernel(x, indices):
    idxt = indices.astype(jnp.int32).T
    out_t = _sc_gather_t(x, idxt)
    return out_t.T
